# baseline (speedup 1.0000x reference)
import jax
import jax.numpy as jnp
from jax import lax
from jax.experimental import pallas as pl
from jax.experimental.pallas import tpu as pltpu
from jax.experimental.pallas import tpu_sc as plsc

N = 10000
E = 320000
D_IN = 128
D_OUT = 16

NC = 2
NS = 16
L = 16

CHUNK = 128
K_CHUNKS = -(-E // (CHUNK * NS))
K_CHUNKS += K_CHUNKS % 2
E_TILE = K_CHUNKS * CHUNK
E_PAD = E_TILE * NS
N_PAD = 10240
ROWS_T = N_PAD // NS
G_TILE = N_PAD // (NC * NS)
G_CHUNK = 64
G_STEPS = G_TILE // G_CHUNK


def _leaky(x):
    return jnp.where(x >= 0, x, 0.2 * x)


def _row_dot(att, xp):
    return lax.dot_general(att, xp, (((1,), (1,)), ((), ())),
                           precision=lax.Precision.HIGHEST,
                           preferred_element_type=jnp.float32)


def _dense_body(x1, x2, w1, w2, s1, d1, s2, d2,
                xp1_o, xp2_o, as1_o, ad1_o, m1_o, as2_o, ad2_o, m2_o):
    zpad = jnp.zeros((N_PAD - N, D_OUT), jnp.float32)
    xp1 = jnp.concatenate(
        [jnp.dot(x1[...], w1[...], preferred_element_type=jnp.float32), zpad])
    xp2 = jnp.concatenate(
        [jnp.dot(x2[...], w2[...], preferred_element_type=jnp.float32), zpad])
    xp1_o[...] = xp1
    xp2_o[...] = xp2
    as1 = _row_dot(s1[...], xp1)
    ad1 = _row_dot(d1[...], xp1)
    as2 = _row_dot(s2[...], xp2)
    ad2 = _row_dot(d2[...], xp2)
    as1_o[...] = as1
    ad1_o[...] = ad1
    as2_o[...] = as2
    ad2_o[...] = ad2
    m1_o[...] = _leaky(jnp.max(as1) + ad1)
    m2_o[...] = _leaky(jnp.max(as2) + ad2)


def _dense_call(x1, x2, w1, w2, s1, d1, s2, d2):
    f32 = jnp.float32
    out_shape = [jax.ShapeDtypeStruct((N_PAD, D_OUT), f32)] * 2 + \
                [jax.ShapeDtypeStruct((1, N_PAD), f32)] * 6
    return pl.pallas_call(_dense_body, out_shape=out_shape)(
        x1, x2, w1, w2, s1.reshape(1, D_OUT), d1.reshape(1, D_OUT),
        s2.reshape(1, D_OUT), d2.reshape(1, D_OUT))


def _gat_on_core(s, ei_h, as_h, ad_h, m_h, xp_h, b_h, out_h,
                 as_t, ad_t, m_t, den_t, den16, idx2_b, e_b, rows_b, rbuf,
                 bias_v, acc, den_stage, sem_i, sem_g):
    iota16 = lax.iota(jnp.int32, L)

    pltpu.sync_copy(as_h.at[0], as_t)
    pltpu.sync_copy(ad_h.at[0], ad_t)
    pltpu.sync_copy(m_h.at[0], m_t)
    pltpu.sync_copy(b_h, bias_v)

    vbase = s * ROWS_T
    pltpu.sync_copy(xp_h.at[pl.ds(vbase, ROWS_T)], rbuf)

    def _zero_den(j, _):
        den_t[pl.ds(j * L, L)] = jnp.zeros((L,), jnp.float32)
        return 0
    lax.fori_loop(0, N_PAD // L, _zero_den, 0)

    def _self(jj, _):
        v0 = vbase + jj * L
        a = as_t[pl.ds(v0, L)] + ad_t[pl.ds(v0, L)]
        a = jnp.where(a >= 0, a, 0.2 * a)
        ev = jnp.exp(a - m_t[pl.ds(v0, L)])
        den_t[pl.ds(v0, L)] = ev
        for i in range(L):
            r = jj * L + i
            rbuf[r, :] = rbuf[r, :] * ev[i]
        return 0
    lax.fori_loop(0, ROWS_T // L, _self, 0)
    pltpu.sync_copy(rbuf, acc.at[pl.ds(vbase, ROWS_T)])
    plsc.subcore_barrier()

    base0 = s * E_TILE

    def phase(k, p, issue_next, prefetch_idx):
        if issue_next:
            pltpu.make_async_copy(ei_h.at[:, pl.ds(0, CHUNK)], idx2_b[1 - p],
                                  sem_i).wait()
            pltpu.async_copy(xp_h.at[idx2_b[1 - p].at[0]], rows_b[1 - p],
                             sem_g[1 - p])

        raw = base0 + k * CHUNK

        def _evec(j, _):
            si = idx2_b[p][0, pl.ds(j * L, L)]
            di = idx2_b[p][1, pl.ds(j * L, L)]
            a = plsc.load_gather(as_t, [si]) + plsc.load_gather(ad_t, [di])
            a = jnp.where(a >= 0, a, 0.2 * a)
            e = jnp.exp(a - plsc.load_gather(m_t, [di]))
            gid = jnp.minimum(raw, E - CHUNK) + j * L + iota16
            e = jnp.where(gid >= raw, e, 0.0)
            e_b[p][pl.ds(j * L, L)] = e
            plsc.addupdate_scatter(den_t, [di], e)
            return 0
        lax.fori_loop(0, CHUNK // L, _evec, 0, unroll=True)

        pltpu.make_async_copy(xp_h.at[pl.ds(0, CHUNK)], rows_b[p],
                              sem_g[p]).wait()

        def _scale(jj, _):
            w16 = e_b[p][pl.ds(jj * L, L)]
            for i in range(L):
                r = jj * L + i
                rows_b[p][r, :] = rows_b[p][r, :] * w16[i]
            return 0
        lax.fori_loop(0, CHUNK // L, _scale, 0)
        pltpu.sync_copy(rows_b[p], acc.at[idx2_b[p].at[1]], add=True)

        if prefetch_idx:
            b2 = jnp.minimum(raw + 2 * CHUNK, E - CHUNK)
            pltpu.async_copy(ei_h.at[:, pl.ds(b2, CHUNK)], idx2_b[p], sem_i)

    pltpu.sync_copy(ei_h.at[:, pl.ds(jnp.minimum(base0, E - CHUNK), CHUNK)],
                    idx2_b[0])
    pltpu.async_copy(xp_h.at[idx2_b[0].at[0]], rows_b[0], sem_g[0])
    pltpu.async_copy(
        ei_h.at[:, pl.ds(jnp.minimum(base0 + CHUNK, E - CHUNK), CHUNK)],
        idx2_b[1], sem_i)

    def _pair(k2, _):
        k = k2 * 2
        phase(k, 0, True, True)
        phase(k + 1, 1, True, True)
        return 0
    lax.fori_loop(0, K_CHUNKS // 2 - 1, _pair, 0)
    phase(K_CHUNKS - 2, 0, True, False)
    phase(K_CHUNKS - 1, 1, False, False)

    pltpu.sync_copy(den_t, den_stage.at[s])
    plsc.subcore_barrier()
    for t in range(NS):
        pltpu.sync_copy(den_stage.at[t, pl.ds(vbase, ROWS_T)], den16.at[t])

    pltpu.sync_copy(acc.at[pl.ds(vbase, ROWS_T)], rbuf)
    bias = bias_v[...]

    def _norm(jj, _):
        d = den16[0, pl.ds(jj * L, L)]
        for t in range(1, NS):
            d = d + den16[t, pl.ds(jj * L, L)]
        inv = 1.0 / d
        for i in range(L):
            r = jj * L + i
            rbuf[r, :] = rbuf[r, :] * inv[i] + bias
        return 0
    lax.fori_loop(0, ROWS_T // L, _norm, 0)
    pltpu.sync_copy(rbuf, out_h.at[pl.ds(vbase, ROWS_T)])


def _edge_body(ei1, ei2, as1, ad1, m1, as2, ad2, m2, xp1, xp2, b1, b2,
               x1o, x2o,
               as_t, ad_t, m_t, den_t, den16, idx2_0, idx2_1, e_c0, e_c1,
               rows0, rows1, rbuf, bias_v, acc, den_stage,
               sem_i, sem_g0, sem_g1):
    c = lax.axis_index("c")
    s = lax.axis_index("s")
    idx2_b = (idx2_0, idx2_1)
    e_b = (e_c0, e_c1)
    rows_b = (rows0, rows1)
    sem_g = (sem_g0, sem_g1)

    @pl.when(c == 0)
    def _():
        _gat_on_core(s, ei1, as1, ad1, m1, xp1, b1, x1o,
                     as_t, ad_t, m_t, den_t, den16, idx2_b, e_b, rows_b,
                     rbuf, bias_v, acc, den_stage, sem_i, sem_g)

    @pl.when(c == 1)
    def _():
        _gat_on_core(s, ei2, as2, ad2, m2, xp2, b2, x2o,
                     as_t, ad_t, m_t, den_t, den16, idx2_b, e_b, rows_b,
                     rbuf, bias_v, acc, den_stage, sem_i, sem_g)


def _edge_call(ei1, ei2, as1, ad1, m1, as2, ad2, m2, xp1, xp2, b1, b2):
    f32 = jnp.float32
    mesh = plsc.VectorSubcoreMesh(core_axis_name="c", subcore_axis_name="s")
    out_type = [
        jax.ShapeDtypeStruct((N_PAD, D_OUT), f32),
        jax.ShapeDtypeStruct((N_PAD, D_OUT), f32),
    ]
    scratch = [
        pltpu.VMEM((N_PAD,), f32),
        pltpu.VMEM((N_PAD,), f32),
        pltpu.VMEM((N_PAD,), f32),
        pltpu.VMEM((N_PAD,), f32),
        pltpu.VMEM((NS, ROWS_T), f32),
        pltpu.VMEM((2, CHUNK), jnp.int32),
        pltpu.VMEM((2, CHUNK), jnp.int32),
        pltpu.VMEM((CHUNK,), f32),
        pltpu.VMEM((CHUNK,), f32),
        pltpu.VMEM((CHUNK, D_OUT), f32),
        pltpu.VMEM((CHUNK, D_OUT), f32),
        pltpu.VMEM((ROWS_T, D_OUT), f32),
        pltpu.VMEM((L,), f32),
        pltpu.VMEM_SHARED((N_PAD, D_OUT), f32),
        pltpu.VMEM_SHARED((NS, N_PAD), f32),
        pltpu.SemaphoreType.DMA,
        pltpu.SemaphoreType.DMA,
        pltpu.SemaphoreType.DMA,
    ]
    kfn = pl.kernel(_edge_body, out_type=out_type, mesh=mesh,
                    scratch_types=scratch,
                    compiler_params=pltpu.CompilerParams(
                        needs_layout_passes=False,
                        use_tc_tiling_on_sc=False))
    return kfn(ei1, ei2, as1, ad1, m1, as2, ad2, m2, xp1, xp2, b1, b2)


def _gather_body(x1o, x2o, grp, out, idx_v, gbuf, xbuf, sem):
    c = lax.axis_index("c")
    s = lax.axis_index("s")
    base = (s * NC + c) * G_TILE

    def _step(k, _):
        off = jnp.minimum(base + k * G_CHUNK, N - G_CHUNK)
        pltpu.sync_copy(grp.at[pl.ds(off, G_CHUNK)], idx_v)
        cp = pltpu.async_copy(x2o.at[idx_v], gbuf, sem)
        pltpu.sync_copy(x1o.at[pl.ds(off, G_CHUNK)], xbuf)
        cp.wait()

        def _add(j, _):
            xbuf[j, :] = xbuf[j, :] + gbuf[j, :]
            return 0
        lax.fori_loop(0, G_CHUNK, _add, 0, unroll=8)
        pltpu.sync_copy(xbuf, out.at[pl.ds(off, G_CHUNK)])
        return 0
    lax.fori_loop(0, G_STEPS, _step, 0)


def _gather_call(x1o, x2o, grp):
    mesh = plsc.VectorSubcoreMesh(core_axis_name="c", subcore_axis_name="s")
    out_type = jax.ShapeDtypeStruct((N, D_OUT), jnp.float32)
    scratch = [
        pltpu.VMEM((G_CHUNK,), jnp.int32),
        pltpu.VMEM((G_CHUNK, D_OUT), jnp.float32),
        pltpu.VMEM((G_CHUNK, D_OUT), jnp.float32),
        pltpu.SemaphoreType.DMA,
    ]
    kfn = pl.kernel(_gather_body, out_type=out_type, mesh=mesh,
                    scratch_types=scratch,
                    compiler_params=pltpu.CompilerParams(
                        needs_layout_passes=False,
                        use_tc_tiling_on_sc=False))
    return kfn(x1o, x2o, grp)


@jax.jit
def kernel(x1, edge_index1, x2, edge_index2, group_assignment,
           W1, att_src1, att_dst1, b1, W2, att_src2, att_dst2, b2):
    xp1, xp2, as1, ad1, m1, as2, ad2, m2 = _dense_call(
        x1, x2, W1, W2, att_src1, att_dst1, att_src2, att_dst2)

    x1_out, x2_out = _edge_call(
        edge_index1, edge_index2, as1, ad1, m1, as2, ad2, m2, xp1, xp2,
        b1, b2)

    grp = group_assignment.astype(jnp.int32)
    x1_combined = _gather_call(x1_out, x2_out, grp)
    return (x1_combined, x2_out[:N])


# speedup vs baseline: 1.0281x; 1.0281x over previous
import jax
import jax.numpy as jnp
from jax import lax
from jax.experimental import pallas as pl
from jax.experimental.pallas import tpu as pltpu
from jax.experimental.pallas import tpu_sc as plsc

N = 10000
E = 320000
D_IN = 128
D_OUT = 16

NC = 2
NS = 16
L = 16

CHUNK = 128
K_CHUNKS = -(-E // (CHUNK * NS))
K_CHUNKS += K_CHUNKS % 2
E_TILE = K_CHUNKS * CHUNK
E_PAD = E_TILE * NS
N_PAD = 10240
ROWS_T = N_PAD // NS
G_TILE = N_PAD // (NC * NS)
G_CHUNK = 64
G_STEPS = G_TILE // G_CHUNK


def _leaky(x):
    return jnp.where(x >= 0, x, 0.2 * x)


def _row_dot(att, xp):
    return lax.dot_general(att, xp, (((1,), (1,)), ((), ())),
                           precision=lax.Precision.HIGHEST,
                           preferred_element_type=jnp.float32)


def _dense_body(x1, x2, w1, w2, s1, d1, s2, d2,
                xp1_o, xp2_o, aa1_o, m1_o, aa2_o, m2_o):
    zpad = jnp.zeros((N_PAD - N, D_OUT), jnp.float32)
    xp1 = jnp.concatenate(
        [jnp.dot(x1[...], w1[...], preferred_element_type=jnp.float32), zpad])
    xp2 = jnp.concatenate(
        [jnp.dot(x2[...], w2[...], preferred_element_type=jnp.float32), zpad])
    xp1_o[...] = xp1
    xp2_o[...] = xp2
    aa1 = _row_dot(jnp.concatenate([s1[...], d1[...]]), xp1)
    aa2 = _row_dot(jnp.concatenate([s2[...], d2[...]]), xp2)
    aa1_o[...] = aa1
    aa2_o[...] = aa2
    m1_o[...] = _leaky(jnp.max(aa1[0:1]) + aa1[1:2])
    m2_o[...] = _leaky(jnp.max(aa2[0:1]) + aa2[1:2])


def _dense_call(x1, x2, w1, w2, s1, d1, s2, d2):
    f32 = jnp.float32
    out_shape = [jax.ShapeDtypeStruct((N_PAD, D_OUT), f32)] * 2 + \
                [jax.ShapeDtypeStruct((2, N_PAD), f32),
                 jax.ShapeDtypeStruct((1, N_PAD), f32)] * 2
    return pl.pallas_call(_dense_body, out_shape=out_shape)(
        x1, x2, w1, w2, s1.reshape(1, D_OUT), d1.reshape(1, D_OUT),
        s2.reshape(1, D_OUT), d2.reshape(1, D_OUT))


def _gat_on_core(s, ei_h, aa_h, m_h, xp_h, b_h, out_h,
                 as_t, ad_t, m_t, den_t, den16, idx2_b, e_b, rows_b, rbuf,
                 bias_v, acc, den_stage, sem_i, sem_g):
    iota16 = lax.iota(jnp.int32, L)

    pltpu.sync_copy(aa_h.at[0], as_t)
    pltpu.sync_copy(aa_h.at[1], ad_t)
    pltpu.sync_copy(m_h.at[0], m_t)
    pltpu.sync_copy(b_h, bias_v)

    vbase = s * ROWS_T
    pltpu.sync_copy(xp_h.at[pl.ds(vbase, ROWS_T)], rbuf)

    def _zero_den(j, _):
        den_t[pl.ds(j * L, L)] = jnp.zeros((L,), jnp.float32)
        return 0
    lax.fori_loop(0, N_PAD // L, _zero_den, 0)

    def _self(jj, _):
        v0 = vbase + jj * L
        a = as_t[pl.ds(v0, L)] + ad_t[pl.ds(v0, L)]
        a = jnp.where(a >= 0, a, 0.2 * a)
        ev = jnp.exp(a - m_t[pl.ds(v0, L)])
        den_t[pl.ds(v0, L)] = ev
        for i in range(L):
            r = jj * L + i
            rbuf[r, :] = rbuf[r, :] * ev[i]
        return 0
    lax.fori_loop(0, ROWS_T // L, _self, 0)
    pltpu.sync_copy(rbuf, acc.at[pl.ds(vbase, ROWS_T)])
    plsc.subcore_barrier()

    base0 = s * E_TILE

    def phase(k, p, issue_next, prefetch_idx):
        if issue_next:
            pltpu.make_async_copy(ei_h.at[:, pl.ds(0, CHUNK)], idx2_b[1 - p],
                                  sem_i).wait()
            pltpu.async_copy(xp_h.at[idx2_b[1 - p].at[0]], rows_b[1 - p],
                             sem_g[1 - p])

        raw = base0 + k * CHUNK

        def _evec(j, _):
            si = idx2_b[p][0, pl.ds(j * L, L)]
            di = idx2_b[p][1, pl.ds(j * L, L)]
            a = plsc.load_gather(as_t, [si]) + plsc.load_gather(ad_t, [di])
            a = jnp.where(a >= 0, a, 0.2 * a)
            e = jnp.exp(a - plsc.load_gather(m_t, [di]))
            gid = jnp.minimum(raw, E - CHUNK) + j * L + iota16
            e = jnp.where(gid >= raw, e, 0.0)
            e_b[p][pl.ds(j * L, L)] = e
            plsc.addupdate_scatter(den_t, [di], e)
            return 0
        lax.fori_loop(0, CHUNK // L, _evec, 0, unroll=True)

        pltpu.make_async_copy(xp_h.at[pl.ds(0, CHUNK)], rows_b[p],
                              sem_g[p]).wait()

        def _scale(jj, _):
            w16 = e_b[p][pl.ds(jj * L, L)]
            for i in range(L):
                r = jj * L + i
                rows_b[p][r, :] = rows_b[p][r, :] * w16[i]
            return 0
        lax.fori_loop(0, CHUNK // L, _scale, 0)
        pltpu.sync_copy(rows_b[p], acc.at[idx2_b[p].at[1]], add=True)

        if prefetch_idx:
            b2 = jnp.minimum(raw + 2 * CHUNK, E - CHUNK)
            pltpu.async_copy(ei_h.at[:, pl.ds(b2, CHUNK)], idx2_b[p], sem_i)

    pltpu.sync_copy(ei_h.at[:, pl.ds(jnp.minimum(base0, E - CHUNK), CHUNK)],
                    idx2_b[0])
    pltpu.async_copy(xp_h.at[idx2_b[0].at[0]], rows_b[0], sem_g[0])
    pltpu.async_copy(
        ei_h.at[:, pl.ds(jnp.minimum(base0 + CHUNK, E - CHUNK), CHUNK)],
        idx2_b[1], sem_i)

    def _pair(k2, _):
        k = k2 * 2
        phase(k, 0, True, True)
        phase(k + 1, 1, True, True)
        return 0
    lax.fori_loop(0, K_CHUNKS // 2 - 1, _pair, 0)
    phase(K_CHUNKS - 2, 0, True, False)
    phase(K_CHUNKS - 1, 1, False, False)

    pltpu.sync_copy(den_t, den_stage.at[s])
    plsc.subcore_barrier()
    for t in range(NS):
        pltpu.sync_copy(den_stage.at[t, pl.ds(vbase, ROWS_T)], den16.at[t])

    pltpu.sync_copy(acc.at[pl.ds(vbase, ROWS_T)], rbuf)
    bias = bias_v[...]

    def _norm(jj, _):
        d = den16[0, pl.ds(jj * L, L)]
        for t in range(1, NS):
            d = d + den16[t, pl.ds(jj * L, L)]
        inv = 1.0 / d
        for i in range(L):
            r = jj * L + i
            rbuf[r, :] = rbuf[r, :] * inv[i] + bias
        return 0
    lax.fori_loop(0, ROWS_T // L, _norm, 0)
    pltpu.sync_copy(rbuf, out_h.at[pl.ds(vbase, ROWS_T)])


def _edge_body(ei1, ei2, aa1, m1, aa2, m2, xp1, xp2, b1, b2,
               x1o, x2o,
               as_t, ad_t, m_t, den_t, den16, idx2_0, idx2_1, e_c0, e_c1,
               rows0, rows1, rbuf, bias_v, acc, den_stage,
               sem_i, sem_g0, sem_g1):
    c = lax.axis_index("c")
    s = lax.axis_index("s")
    idx2_b = (idx2_0, idx2_1)
    e_b = (e_c0, e_c1)
    rows_b = (rows0, rows1)
    sem_g = (sem_g0, sem_g1)

    @pl.when(c == 0)
    def _():
        _gat_on_core(s, ei1, aa1, m1, xp1, b1, x1o,
                     as_t, ad_t, m_t, den_t, den16, idx2_b, e_b, rows_b,
                     rbuf, bias_v, acc, den_stage, sem_i, sem_g)

    @pl.when(c == 1)
    def _():
        _gat_on_core(s, ei2, aa2, m2, xp2, b2, x2o,
                     as_t, ad_t, m_t, den_t, den16, idx2_b, e_b, rows_b,
                     rbuf, bias_v, acc, den_stage, sem_i, sem_g)


def _edge_call(ei1, ei2, aa1, m1, aa2, m2, xp1, xp2, b1, b2):
    f32 = jnp.float32
    mesh = plsc.VectorSubcoreMesh(core_axis_name="c", subcore_axis_name="s")
    out_type = [
        jax.ShapeDtypeStruct((N_PAD, D_OUT), f32),
        jax.ShapeDtypeStruct((N_PAD, D_OUT), f32),
    ]
    scratch = [
        pltpu.VMEM((N_PAD,), f32),
        pltpu.VMEM((N_PAD,), f32),
        pltpu.VMEM((N_PAD,), f32),
        pltpu.VMEM((N_PAD,), f32),
        pltpu.VMEM((NS, ROWS_T), f32),
        pltpu.VMEM((2, CHUNK), jnp.int32),
        pltpu.VMEM((2, CHUNK), jnp.int32),
        pltpu.VMEM((CHUNK,), f32),
        pltpu.VMEM((CHUNK,), f32),
        pltpu.VMEM((CHUNK, D_OUT), f32),
        pltpu.VMEM((CHUNK, D_OUT), f32),
        pltpu.VMEM((ROWS_T, D_OUT), f32),
        pltpu.VMEM((L,), f32),
        pltpu.VMEM_SHARED((N_PAD, D_OUT), f32),
        pltpu.VMEM_SHARED((NS, N_PAD), f32),
        pltpu.SemaphoreType.DMA,
        pltpu.SemaphoreType.DMA,
        pltpu.SemaphoreType.DMA,
    ]
    kfn = pl.kernel(_edge_body, out_type=out_type, mesh=mesh,
                    scratch_types=scratch,
                    compiler_params=pltpu.CompilerParams(
                        needs_layout_passes=False,
                        use_tc_tiling_on_sc=False))
    return kfn(ei1, ei2, aa1, m1, aa2, m2, xp1, xp2, b1, b2)


def _gather_body(x1o, x2o, grp, out, idx_v, gbuf, xbuf, sem):
    c = lax.axis_index("c")
    s = lax.axis_index("s")
    base = (s * NC + c) * G_TILE

    def _step(k, _):
        off = jnp.minimum(base + k * G_CHUNK, N - G_CHUNK)
        pltpu.sync_copy(grp.at[pl.ds(off, G_CHUNK)], idx_v)
        cp = pltpu.async_copy(x2o.at[idx_v], gbuf, sem)
        pltpu.sync_copy(x1o.at[pl.ds(off, G_CHUNK)], xbuf)
        cp.wait()

        def _add(j, _):
            xbuf[j, :] = xbuf[j, :] + gbuf[j, :]
            return 0
        lax.fori_loop(0, G_CHUNK, _add, 0, unroll=8)
        pltpu.sync_copy(xbuf, out.at[pl.ds(off, G_CHUNK)])
        return 0
    lax.fori_loop(0, G_STEPS, _step, 0)


def _gather_call(x1o, x2o, grp):
    mesh = plsc.VectorSubcoreMesh(core_axis_name="c", subcore_axis_name="s")
    out_type = jax.ShapeDtypeStruct((N, D_OUT), jnp.float32)
    scratch = [
        pltpu.VMEM((G_CHUNK,), jnp.int32),
        pltpu.VMEM((G_CHUNK, D_OUT), jnp.float32),
        pltpu.VMEM((G_CHUNK, D_OUT), jnp.float32),
        pltpu.SemaphoreType.DMA,
    ]
    kfn = pl.kernel(_gather_body, out_type=out_type, mesh=mesh,
                    scratch_types=scratch,
                    compiler_params=pltpu.CompilerParams(
                        needs_layout_passes=False,
                        use_tc_tiling_on_sc=False))
    return kfn(x1o, x2o, grp)


@jax.jit
def kernel(x1, edge_index1, x2, edge_index2, group_assignment,
           W1, att_src1, att_dst1, b1, W2, att_src2, att_dst2, b2):
    xp1, xp2, aa1, m1, aa2, m2 = _dense_call(
        x1, x2, W1, W2, att_src1, att_dst1, att_src2, att_dst2)

    x1_out, x2_out = _edge_call(
        edge_index1, edge_index2, aa1, m1, aa2, m2, xp1, xp2, b1, b2)

    grp = group_assignment.astype(jnp.int32)
    x1_combined = _gather_call(x1_out, x2_out, grp)
    return (x1_combined, x2_out[:N])
